# Initial kernel scaffold; baseline (speedup 1.0000x reference)
import jax
import jax.numpy as jnp
from jax.experimental import pallas as pl


def kernel(tokens, token_types, t, W1, b1, W2, b2):
    raise NotImplementedError("write your pallas kernel here")


# speedup vs baseline: 1.6445x; 1.6445x over previous
import functools

import numpy as np
import jax
import jax.numpy as jnp
from jax.experimental import pallas as pl
from jax.experimental.pallas import tpu as pltpu

NUM_C_TYPES = 6
TTYPE_UNKNOWN = 5
E = 8
D = 768
H = D // 4
TEMP = 0.1
SOFT_RES = 0.05
FLOOR = min(0.05, 0.15 / 4)
CAP_LOW = 0.5
CAP_HIGH = 0.6
T_MAX = 1000

TBLK = 1024


def _base_table():
    base = np.zeros((NUM_C_TYPES, E), dtype=np.float32)
    for t_type in range(NUM_C_TYPES - 1):
        base[t_type, t_type % E] = 1.0
    base[TTYPE_UNKNOWN] = 1.0 / E
    num_known = NUM_C_TYPES - 1
    if E > num_known:
        for extra_idx in range(num_known, E):
            paired = extra_idx % num_known
            base[paired, extra_idx] = 0.3
        for t_type in range(NUM_C_TYPES - 1):
            s = base[t_type].sum()
            if s > 0:
                base[t_type] = base[t_type] / s
    return base


def _router_body(t_ref, x_ref, tt_ref, W1_ref, b1_ref, W2_ref, b2_ref,
                 base_ref, out_ref):
    x = x_ref[0]
    tt = tt_ref[0, 0]

    u = jax.lax.dot_general(x, W1_ref[...], (((1,), (0,)), ((), ())),
                            preferred_element_type=jnp.float32) + b1_ref[...]
    h = 0.5 * u * (1.0 + jax.lax.erf(u * np.float32(1.0 / np.sqrt(2.0))))
    logits = (jax.lax.dot_general(h, W2_ref[...], (((1,), (0,)), ((), ())),
                                  preferred_element_type=jnp.float32)
              + b2_ref[...]) * (1.0 / TEMP)

    z = logits - jnp.max(logits, axis=-1, keepdims=True)
    ez = jnp.exp(z)
    p2 = ez / jnp.sum(ez, axis=-1, keepdims=True)

    base = base_ref[...]
    bp = jnp.zeros_like(p2)
    for k in range(NUM_C_TYPES):
        row = jax.lax.slice(base, (k, 0), (k + 1, E))
        bp = bp + jnp.where(tt == k, row, 0.0)

    w = jnp.where(tt == TTYPE_UNKNOWN, 0.0, 1.0 - SOFT_RES)
    blended = w * bp + (1.0 - w) * p2

    alpha = min(FLOOR * E, 1.0)
    probs = (1.0 - alpha) * blended + alpha / E

    b = pl.program_id(0)
    t_norm = t_ref[b].astype(jnp.float32) / T_MAX
    cap = CAP_LOW + (CAP_HIGH + CAP_LOW) * t_norm

    excess = jnp.maximum(probs - cap, 0.0)
    capped = probs - excess
    headroom = jnp.maximum(cap - capped, 0.0)
    hs = jnp.maximum(jnp.sum(headroom, axis=-1, keepdims=True), 1e-8)
    final = capped + jnp.sum(excess, axis=-1, keepdims=True) * (headroom / hs)

    m = jnp.max(final, axis=-1, keepdims=True)
    idx = jax.lax.broadcasted_iota(jnp.int32, final.shape, 1)
    cand = jnp.where(final >= m, idx, E)
    amin = jnp.min(cand, axis=-1, keepdims=True)
    out_ref[0] = (idx == amin).astype(jnp.float32)


@jax.jit
def _router(tokens, token_types, t, W1, b1, W2, b2, base):
    B, N, _ = tokens.shape
    nblk = N // TBLK
    tt4 = token_types.reshape(B, nblk, TBLK, 1)
    grid = (B, nblk)
    onehot = pl.pallas_call(
        _router_body,
        grid=grid,
        in_specs=[
            pl.BlockSpec(memory_space=pltpu.SMEM),
            pl.BlockSpec((1, TBLK, D), lambda b, j: (b, j, 0)),
            pl.BlockSpec((1, 1, TBLK, 1), lambda b, j: (b, j, 0, 0)),
            pl.BlockSpec((D, H), lambda b, j: (0, 0)),
            pl.BlockSpec((1, H), lambda b, j: (0, 0)),
            pl.BlockSpec((H, E), lambda b, j: (0, 0)),
            pl.BlockSpec((1, E), lambda b, j: (0, 0)),
            pl.BlockSpec((NUM_C_TYPES, E), lambda b, j: (0, 0)),
        ],
        out_specs=pl.BlockSpec((1, TBLK, E), lambda b, j: (b, j, 0)),
        out_shape=jax.ShapeDtypeStruct((B, N, E), jnp.float32),
        compiler_params=pltpu.CompilerParams(
            dimension_semantics=("parallel", "parallel")),
    )(t, tokens, tt4, W1, b1.reshape(1, H), W2, b2.reshape(1, E), base)
    return onehot


def kernel(tokens, token_types, t, W1, b1, W2, b2):
    base = jnp.asarray(_base_table())
    onehot = _router(tokens, token_types, t, W1, b1, W2, b2, base)
    return onehot, onehot


# speedup vs baseline: 3.3747x; 2.0521x over previous
import functools

import numpy as np
import jax
import jax.numpy as jnp
from jax.experimental import pallas as pl
from jax.experimental.pallas import tpu as pltpu

NUM_C_TYPES = 6
TTYPE_UNKNOWN = 5
E = 8
D = 768
H = D // 4
TEMP = 0.1
SOFT_RES = 0.05
FLOOR = min(0.05, 0.15 / 4)
CAP_LOW = 0.5
CAP_HIGH = 0.6
T_MAX = 1000

TBLK = 1024


def _base_table():
    base = np.zeros((NUM_C_TYPES, E), dtype=np.float32)
    for t_type in range(NUM_C_TYPES - 1):
        base[t_type, t_type % E] = 1.0
    base[TTYPE_UNKNOWN] = 1.0 / E
    num_known = NUM_C_TYPES - 1
    if E > num_known:
        for extra_idx in range(num_known, E):
            paired = extra_idx % num_known
            base[paired, extra_idx] = 0.3
        for t_type in range(NUM_C_TYPES - 1):
            s = base[t_type].sum()
            if s > 0:
                base[t_type] = base[t_type] / s
    return base


def _router_body(t_ref, x_ref, tt_ref, W1_ref, b1_ref, W2_ref, b2_ref,
                 baseT_ref, eye_ref, out_ref):
    x = x_ref[0]
    tt = tt_ref[0, 0]

    u = jax.lax.dot_general(x, W1_ref[...], (((1,), (0,)), ((), ())),
                            preferred_element_type=jnp.float32) + b1_ref[...]
    h = 0.5 * u * (1.0 + jax.lax.erf(u * np.float32(1.0 / np.sqrt(2.0))))

    logitsT = (jax.lax.dot_general(W2_ref[...], h, (((0,), (1,)), ((), ())),
                                   preferred_element_type=jnp.float32)
               + b2_ref[...]) * (1.0 / TEMP)

    z = logitsT - jnp.max(logitsT, axis=0, keepdims=True)
    ez = jnp.exp(z)
    p2 = ez / jnp.sum(ez, axis=0, keepdims=True)

    baseT = baseT_ref[...]
    bp = jnp.zeros_like(p2)
    for k in range(NUM_C_TYPES):
        col = jax.lax.slice(baseT, (0, k), (E, k + 1))
        bp = bp + jnp.where(tt == k, col, 0.0)

    w = jnp.where(tt == TTYPE_UNKNOWN, 0.0, 1.0 - SOFT_RES)
    blended = w * bp + (1.0 - w) * p2

    alpha = min(FLOOR * E, 1.0)
    probs = (1.0 - alpha) * blended + alpha / E

    b = pl.program_id(0)
    t_norm = t_ref[b].astype(jnp.float32) / T_MAX
    cap = CAP_LOW + (CAP_HIGH + CAP_LOW) * t_norm

    excess = jnp.maximum(probs - cap, 0.0)
    capped = probs - excess
    headroom = jnp.maximum(cap - capped, 0.0)
    hs = jnp.maximum(jnp.sum(headroom, axis=0, keepdims=True), 1e-8)
    final = capped + jnp.sum(excess, axis=0, keepdims=True) * (headroom / hs)

    m = jnp.max(final, axis=0, keepdims=True)
    idx = jax.lax.broadcasted_iota(jnp.int32, final.shape, 0)
    cand = jnp.where(final >= m, idx, E)
    amin = jnp.min(cand, axis=0, keepdims=True)
    yT = (idx == amin).astype(jnp.float32)

    out_ref[0] = jax.lax.dot_general(yT, eye_ref[...], (((0,), (0,)), ((), ())),
                                     preferred_element_type=jnp.float32)


@jax.jit
def _router(tokens, token_types, t, W1, b1, W2, b2, baseT, eye):
    B, N, _ = tokens.shape
    nblk = N // TBLK
    tt4 = token_types.reshape(B, nblk, 1, TBLK)
    grid = (B, nblk)
    onehot = pl.pallas_call(
        _router_body,
        grid=grid,
        in_specs=[
            pl.BlockSpec(memory_space=pltpu.SMEM),
            pl.BlockSpec((1, TBLK, D), lambda b, j: (b, j, 0)),
            pl.BlockSpec((1, 1, 1, TBLK), lambda b, j: (b, j, 0, 0)),
            pl.BlockSpec((D, H), lambda b, j: (0, 0)),
            pl.BlockSpec((1, H), lambda b, j: (0, 0)),
            pl.BlockSpec((H, E), lambda b, j: (0, 0)),
            pl.BlockSpec((E, 1), lambda b, j: (0, 0)),
            pl.BlockSpec((E, NUM_C_TYPES), lambda b, j: (0, 0)),
            pl.BlockSpec((E, E), lambda b, j: (0, 0)),
        ],
        out_specs=pl.BlockSpec((1, TBLK, E), lambda b, j: (b, j, 0)),
        out_shape=jax.ShapeDtypeStruct((B, N, E), jnp.float32),
        compiler_params=pltpu.CompilerParams(
            dimension_semantics=("parallel", "parallel")),
    )(t, tokens, tt4, W1, b1.reshape(1, H), W2, b2.reshape(E, 1),
      baseT, eye)
    return onehot


def kernel(tokens, token_types, t, W1, b1, W2, b2):
    baseT = jnp.asarray(_base_table().T.copy())
    eye = jnp.eye(E, dtype=jnp.float32)
    onehot = _router(tokens, token_types, t, W1, b1, W2, b2, baseT, eye)
    return onehot, onehot
